# baseline (speedup 1.0000x reference)
import functools

import jax
import jax.numpy as jnp
from jax import lax
from jax.experimental import pallas as pl
from jax.experimental.pallas import tpu as pltpu
from jax.experimental.pallas import tpu_sc as plsc

_N1 = 6
_N2 = 3
_D = 128
_ROWS = 200
_NOUT = _N1 * _N2


def _body(idx1_hbm, idx2_hbm, xflat_hbm, out_hbm, idx1_s, idx2_s, sem1, sem2,
          gsem):
    cp1 = pltpu.make_async_copy(idx1_hbm, idx1_s, sem1)
    cp2 = pltpu.make_async_copy(idx2_hbm, idx2_s, sem2)
    cp1.start()
    cp2.start()
    cp1.wait()
    cp2.wait()

    def issue(k, carry):
        i = k // _N2
        j = k - i * _N2
        flat = idx1_s[i] * _ROWS + idx2_s[j]
        pltpu.make_async_copy(
            xflat_hbm.at[pl.ds(flat, 1)], out_hbm.at[pl.ds(k, 1)], gsem
        ).start()
        return carry

    def drain(k, carry):
        pltpu.make_async_copy(
            xflat_hbm.at[pl.ds(0, 1)], out_hbm.at[pl.ds(0, 1)], gsem
        ).wait()
        return carry

    lax.fori_loop(0, _NOUT, issue, 0, unroll=False)
    lax.fori_loop(0, _NOUT, drain, 0, unroll=False)


_sc_gather = functools.partial(
    pl.kernel,
    mesh=plsc.ScalarSubcoreMesh(axis_name="c", num_cores=1),
    out_type=jax.ShapeDtypeStruct((_NOUT, _D), jnp.float32),
    scratch_types=[
        pltpu.SMEM((_N1,), jnp.int32),
        pltpu.SMEM((_N2,), jnp.int32),
        pltpu.SemaphoreType.DMA,
        pltpu.SemaphoreType.DMA,
        pltpu.SemaphoreType.DMA,
    ],
)(_body)


@jax.jit
def kernel(x, index1, index2):
    xflat = x.reshape(-1, _D)
    y = _sc_gather(index1.reshape(_N1), index2, xflat)
    return y.reshape(_N1, _N2, _D)
